# Initial kernel scaffold; baseline (speedup 1.0000x reference)
import jax
import jax.numpy as jnp
from jax.experimental import pallas as pl


def kernel(ids, emb, W1, b1, W2, b2):
    raise NotImplementedError("write your pallas kernel here")


# speedup vs baseline: 1.7133x; 1.7133x over previous
import functools

import jax
import jax.numpy as jnp
from jax import lax
from jax.experimental import pallas as pl
from jax.experimental.pallas import tpu as pltpu
from jax.experimental.pallas import tpu_sc as plsc

NC = 2
NS = 16
NW = NC * NS
LANES = 16


@functools.lru_cache(maxsize=None)
def _make_sc_pool(B, L, EMB, V, chunk):
    assert EMB == 2 * LANES
    assert L % chunk == 0 and chunk <= 128
    cpr = L // chunk
    b_per_w = B // NW
    n_chunks = b_per_w * cpr

    mesh = plsc.VectorSubcoreMesh(
        core_axis_name="c", subcore_axis_name="s", num_cores=NC, num_subcores=NS
    )

    @functools.partial(
        pl.kernel,
        out_type=jax.ShapeDtypeStruct((B, EMB), jnp.float32),
        mesh=mesh,
        compiler_params=pltpu.CompilerParams(use_tc_tiling_on_sc=False),
        scratch_types=[
            pltpu.VMEM((n_chunks, chunk), jnp.int32),
            pltpu.VMEM((chunk, EMB), jnp.float32),
            pltpu.VMEM((b_per_w, EMB), jnp.float32),
        ],
    )
    def sc_pool(ids_hbm, table_hbm, pooled_hbm, ids_v, buf, acc_v):
        wid = lax.axis_index("s") * NC + lax.axis_index("c")
        pltpu.sync_copy(ids_hbm.at[wid], ids_v)

        def row_body(r, _):
            def chunk_body(c, carry):
                a0, a1 = carry
                j = r * cpr + c
                pltpu.sync_copy(table_hbm.at[ids_v.at[j]], buf)

                def l_body(l, carry2):
                    b0, b1 = carry2
                    return (b0 + buf[l, pl.ds(0, LANES)],
                            b1 + buf[l, pl.ds(LANES, LANES)])

                return lax.fori_loop(0, chunk, l_body, (a0, a1))

            z = jnp.zeros((LANES,), jnp.float32)
            a0, a1 = lax.fori_loop(0, cpr, chunk_body, (z, z))
            acc_v[r, pl.ds(0, LANES)] = a0
            acc_v[r, pl.ds(LANES, LANES)] = a1
            return 0

        lax.fori_loop(0, b_per_w, row_body, 0)
        pltpu.sync_copy(acc_v, pooled_hbm.at[pl.ds(wid * b_per_w, b_per_w)])

    return sc_pool


def _mlp_body(pooled_ref, ids_ref, W1_ref, b1_ref, W2_ref, b2_ref, out_ref):
    cnt = jnp.sum((ids_ref[...] != 0).astype(jnp.float32), axis=1, keepdims=True)
    avg = pooled_ref[...] / jnp.maximum(cnt, 1e-6)
    h = jnp.maximum(
        jnp.dot(avg, W1_ref[...], preferred_element_type=jnp.float32) + b1_ref[...],
        0.0,
    )
    out_ref[...] = (
        jnp.dot(h, W2_ref[...], preferred_element_type=jnp.float32) + b2_ref[...]
    )


@functools.lru_cache(maxsize=None)
def _make_mlp(B, L, EMB, HID, OUT):
    return pl.pallas_call(
        _mlp_body,
        out_shape=jax.ShapeDtypeStruct((B, OUT), jnp.float32),
        in_specs=[
            pl.BlockSpec(memory_space=pltpu.VMEM),
            pl.BlockSpec(memory_space=pltpu.VMEM),
            pl.BlockSpec(memory_space=pltpu.VMEM),
            pl.BlockSpec(memory_space=pltpu.VMEM),
            pl.BlockSpec(memory_space=pltpu.VMEM),
            pl.BlockSpec(memory_space=pltpu.VMEM),
        ],
        out_specs=pl.BlockSpec(memory_space=pltpu.VMEM),
    )


@jax.jit
def kernel(ids, emb, W1, b1, W2, b2):
    B, L = ids.shape
    V, EMB = emb.shape
    HID = W1.shape[1]
    OUT = W2.shape[1]
    chunk = 100

    ids32 = ids.astype(jnp.int32)
    ids_r = ids32.reshape(NW, (B // NW) * (L // chunk), chunk)

    pooled = _make_sc_pool(B, L, EMB, V, chunk)(ids_r, emb)
    out = _make_mlp(B, L, EMB, HID, OUT)(
        pooled, ids32, W1, b1.reshape(1, HID), W2, b2.reshape(1, OUT)
    )
    return out


# speedup vs baseline: 2.2155x; 1.2931x over previous
import functools

import jax
import jax.numpy as jnp
from jax import lax
from jax.experimental import pallas as pl
from jax.experimental.pallas import tpu as pltpu
from jax.experimental.pallas import tpu_sc as plsc

NC = 2
NS = 16
NW = NC * NS
LANES = 16
C0 = 128


@functools.lru_cache(maxsize=None)
def _make_sc_pool(B, L, EMB, V):
    assert EMB == 2 * LANES
    assert B % NW == 0
    b_per_w = B // NW
    assert b_per_w % 2 == 0
    nq = b_per_w // 2
    C1 = L - C0
    assert 0 < C1 <= 128

    mesh = plsc.VectorSubcoreMesh(
        core_axis_name="c", subcore_axis_name="s", num_cores=NC, num_subcores=NS
    )

    @functools.partial(
        pl.kernel,
        out_type=jax.ShapeDtypeStruct((B, EMB), jnp.float32),
        mesh=mesh,
        compiler_params=pltpu.CompilerParams(use_tc_tiling_on_sc=False),
        scratch_types=[
            pltpu.VMEM((b_per_w, L), jnp.int32),
            pltpu.VMEM((L, EMB), jnp.float32),
            pltpu.VMEM((L, EMB), jnp.float32),
            pltpu.VMEM((b_per_w, EMB), jnp.float32),
            pltpu.SemaphoreType.DMA,
            pltpu.SemaphoreType.DMA,
        ],
    )
    def sc_pool(ids_hbm, table_hbm, pooled_hbm, ids_v, buf0, buf1, acc_v, s0, s1):
        wid = lax.axis_index("s") * NC + lax.axis_index("c")
        base = wid * b_per_w
        pltpu.sync_copy(ids_hbm.at[pl.ds(base, b_per_w)], ids_v)

        bufs = (buf0, buf1)
        sems = (s0, s1)

        def issue(r, p):
            pltpu.async_copy(
                table_hbm.at[ids_v.at[r, pl.ds(0, C0)]],
                bufs[p].at[pl.ds(0, C0)],
                sems[p],
            )
            pltpu.async_copy(
                table_hbm.at[ids_v.at[r, pl.ds(C0, C1)]],
                bufs[p].at[pl.ds(C0, C1)],
                sems[p],
            )

        def wait(p):
            pltpu.make_async_copy(
                table_hbm.at[pl.ds(0, C0)], bufs[p].at[pl.ds(0, C0)], sems[p]
            ).wait()
            pltpu.make_async_copy(
                table_hbm.at[pl.ds(0, C1)], bufs[p].at[pl.ds(C0, C1)], sems[p]
            ).wait()

        def accum(r, buf):
            z = jnp.zeros((LANES,), jnp.float32)
            a0, a1, b0, b1 = z, z, z, z
            for t in range(0, L, 2):
                a0 = a0 + buf[t, pl.ds(0, LANES)]
                a1 = a1 + buf[t, pl.ds(LANES, LANES)]
                b0 = b0 + buf[t + 1, pl.ds(0, LANES)]
                b1 = b1 + buf[t + 1, pl.ds(LANES, LANES)]
            acc_v[r, pl.ds(0, LANES)] = a0 + b0
            acc_v[r, pl.ds(LANES, LANES)] = a1 + b1

        issue(0, 0)

        def body(q, _):
            r0 = 2 * q
            issue(r0 + 1, 1)
            wait(0)
            accum(r0, buf0)

            @pl.when(q < nq - 1)
            def _():
                issue(r0 + 2, 0)

            wait(1)
            accum(r0 + 1, buf1)
            return 0

        lax.fori_loop(0, nq, body, 0)
        pltpu.sync_copy(acc_v, pooled_hbm.at[pl.ds(base, b_per_w)])

    return sc_pool


def _mlp_body(pooled_ref, ids_ref, W1_ref, b1_ref, W2_ref, b2_ref, out_ref):
    cnt = jnp.sum((ids_ref[...] != 0).astype(jnp.float32), axis=1, keepdims=True)
    avg = pooled_ref[...] / jnp.maximum(cnt, 1e-6)
    h = jnp.maximum(
        jnp.dot(avg, W1_ref[...], preferred_element_type=jnp.float32) + b1_ref[...],
        0.0,
    )
    out_ref[...] = (
        jnp.dot(h, W2_ref[...], preferred_element_type=jnp.float32) + b2_ref[...]
    )


@functools.lru_cache(maxsize=None)
def _make_mlp(B, L, EMB, HID, OUT):
    return pl.pallas_call(
        _mlp_body,
        out_shape=jax.ShapeDtypeStruct((B, OUT), jnp.float32),
        in_specs=[
            pl.BlockSpec(memory_space=pltpu.VMEM),
            pl.BlockSpec(memory_space=pltpu.VMEM),
            pl.BlockSpec(memory_space=pltpu.VMEM),
            pl.BlockSpec(memory_space=pltpu.VMEM),
            pl.BlockSpec(memory_space=pltpu.VMEM),
            pl.BlockSpec(memory_space=pltpu.VMEM),
        ],
        out_specs=pl.BlockSpec(memory_space=pltpu.VMEM),
    )


@jax.jit
def kernel(ids, emb, W1, b1, W2, b2):
    B, L = ids.shape
    V, EMB = emb.shape
    HID = W1.shape[1]
    OUT = W2.shape[1]

    ids32 = ids.astype(jnp.int32)
    pooled = _make_sc_pool(B, L, EMB, V)(ids32, emb)
    out = _make_mlp(B, L, EMB, HID, OUT)(
        pooled, ids32, W1, b1.reshape(1, HID), W2, b2.reshape(1, OUT)
    )
    return out
